# baseline (speedup 1.0000x reference)
import functools

import jax
import jax.numpy as jnp
from jax import lax
from jax.experimental import pallas as pl
from jax.experimental.pallas import tpu as pltpu
from jax.experimental.pallas import tpu_sc as plsc

CHUNK = 128


@functools.lru_cache(maxsize=None)
def _build(n_rows: int, dim: int):
    info = plsc.get_sparse_core_info()
    nw = info.num_cores * info.num_subcores
    per_w = n_rows // nw
    n_chunks = per_w // CHUNK
    assert per_w * nw == n_rows and n_chunks * CHUNK == per_w

    mesh = plsc.VectorSubcoreMesh(core_axis_name="c", subcore_axis_name="s")

    @functools.partial(
        pl.kernel,
        mesh=mesh,
        out_type=jax.ShapeDtypeStruct((n_rows, dim), jnp.float32),
        scratch_types=[
            pltpu.VMEM((n_chunks, CHUNK), jnp.int32),
            pltpu.VMEM((CHUNK, dim), jnp.float32),
            pltpu.SemaphoreType.DMA,
        ],
        compiler_params=pltpu.CompilerParams(use_tc_tiling_on_sc=False),
    )
    def emb_kernel(x_hbm, emb_hbm, out_hbm, idx_v, rows_v, gsem):
        wid = lax.axis_index("s") * info.num_cores + lax.axis_index("c")
        base = wid * per_w
        pltpu.sync_copy(x_hbm.at[wid], idx_v)

        def body(j, carry):
            pltpu.async_copy(emb_hbm.at[idx_v.at[j]], rows_v, gsem).wait()
            pltpu.sync_copy(rows_v, out_hbm.at[pl.ds(base + j * CHUNK, CHUNK)])
            return carry

        lax.fori_loop(0, n_chunks, body, 0)

    return emb_kernel, nw, n_chunks


def kernel(x, emb):
    bsz, hist = x.shape
    _, dim = emb.shape
    n_rows = bsz * hist
    emb_kernel, nw, n_chunks = _build(n_rows, dim)
    xr = x.reshape(nw, n_chunks, CHUNK).astype(jnp.int32)
    out = emb_kernel(xr, emb)
    return out.reshape(bsz, hist, dim)


# speedup vs baseline: 1.1130x; 1.1130x over previous
import functools

import jax
import jax.numpy as jnp
from jax import lax
from jax.experimental import pallas as pl
from jax.experimental.pallas import tpu as pltpu
from jax.experimental.pallas import tpu_sc as plsc

CHUNK = 128
NBUF = 12


@functools.lru_cache(maxsize=None)
def _build(n_rows: int, dim: int):
    info = plsc.get_sparse_core_info()
    nw = info.num_cores * info.num_subcores
    per_w = n_rows // nw
    n_chunks = per_w // CHUNK
    assert per_w * nw == n_rows and n_chunks * CHUNK == per_w
    assert n_chunks >= NBUF

    mesh = plsc.VectorSubcoreMesh(core_axis_name="c", subcore_axis_name="s")

    @functools.partial(
        pl.kernel,
        mesh=mesh,
        out_type=jax.ShapeDtypeStruct((n_rows, dim), jnp.float32),
        scratch_types=[
            pltpu.VMEM((n_chunks, CHUNK), jnp.int32),
            pltpu.VMEM((NBUF, CHUNK, dim), jnp.float32),
            pltpu.SemaphoreType.DMA,
            pltpu.SemaphoreType.DMA,
        ],
        compiler_params=pltpu.CompilerParams(use_tc_tiling_on_sc=False),
    )
    def emb_kernel(x_hbm, emb_hbm, out_hbm, idx_v, rows_v, gsem, ssem):
        wid = lax.axis_index("s") * info.num_cores + lax.axis_index("c")
        base = wid * per_w
        pltpu.sync_copy(x_hbm.at[wid], idx_v)

        for b in range(NBUF - 1):
            pltpu.async_copy(emb_hbm.at[idx_v.at[b]], rows_v.at[b], gsem)

        def body(j, carry):
            f = j + NBUF - 1

            @pl.when(f < n_chunks)
            def _():
                @pl.when(j > 0)
                def _():
                    pltpu.make_async_copy(
                        rows_v.at[0], out_hbm.at[pl.ds(base, CHUNK)], ssem
                    ).wait()

                pltpu.async_copy(
                    emb_hbm.at[idx_v.at[f]], rows_v.at[lax.rem(f, NBUF)], gsem
                )

            pltpu.make_async_copy(
                emb_hbm.at[idx_v.at[j]], rows_v.at[lax.rem(j, NBUF)], gsem
            ).wait()
            pltpu.async_copy(
                rows_v.at[lax.rem(j, NBUF)],
                out_hbm.at[pl.ds(base + j * CHUNK, CHUNK)],
                ssem,
            )
            return carry

        lax.fori_loop(0, n_chunks, body, 0)

        def drain(i, carry):
            pltpu.make_async_copy(
                rows_v.at[0], out_hbm.at[pl.ds(base, CHUNK)], ssem
            ).wait()
            return carry

        lax.fori_loop(0, NBUF, drain, 0)

    return emb_kernel, nw, n_chunks


def kernel(x, emb):
    bsz, hist = x.shape
    _, dim = emb.shape
    n_rows = bsz * hist
    emb_kernel, nw, n_chunks = _build(n_rows, dim)
    xr = x.reshape(nw, n_chunks, CHUNK).astype(jnp.int32)
    out = emb_kernel(xr, emb)
    return out.reshape(bsz, hist, dim)
